# baseline (speedup 1.0000x reference)
import functools

import jax
import jax.numpy as jnp
from jax import lax
from jax.experimental import pallas as pl
from jax.experimental.pallas import tpu as pltpu
from jax.experimental.pallas import tpu_sc as plsc

N = 10000
NP = 10240
E = 320000
NC = 2
NS = 16
NW = NC * NS
C = 80
NCHUNK = 125
EPW = NCHUNK * C
RPS = NP // NS

_mesh = plsc.VectorSubcoreMesh(core_axis_name="c", subcore_axis_name="s")


@functools.partial(
    pl.kernel,
    out_type=jax.ShapeDtypeStruct((NC, NP, 128), jnp.float32),
    mesh=_mesh,
    scratch_types=[
        pltpu.VMEM((NCHUNK, C), jnp.int32),
        pltpu.VMEM((C, 128), jnp.float32),
        pltpu.VMEM_SHARED((NP, 128), jnp.float32),
        pltpu.SemaphoreType.DMA,
    ],
)
def _deg_kernel(dst_hbm, ones_hbm, z_hbm, out_hbm, dst_v, ones_v, acc_sh, sem):
    c = lax.axis_index("c")
    s = lax.axis_index("s")
    w = c * NS + s
    pltpu.sync_copy(z_hbm, acc_sh.at[pl.ds(s * RPS, RPS)])
    pltpu.sync_copy(ones_hbm, ones_v)
    pltpu.sync_copy(dst_hbm.at[w], dst_v)
    plsc.subcore_barrier()

    def body(i, carry):
        pltpu.async_copy(ones_v, acc_sh.at[dst_v.at[i]], sem, add=True)
        return carry

    lax.fori_loop(0, NCHUNK, body, 0)

    def drain(i, carry):
        pltpu.make_async_copy(ones_v, acc_sh.at[dst_v.at[0]], sem).wait()
        return carry

    lax.fori_loop(0, NCHUNK, drain, 0)
    plsc.subcore_barrier()
    pltpu.sync_copy(acc_sh.at[pl.ds(s * RPS, RPS)],
                    out_hbm.at[c, pl.ds(s * RPS, RPS)])


def _make_msg_kernel(K, tc_tiling=True):
  cp = pltpu.CompilerParams(use_tc_tiling_on_sc=tc_tiling)

  @functools.partial(
      pl.kernel,
      out_type=jax.ShapeDtypeStruct((NC, NP, K), jnp.float32),
      mesh=_mesh,
      compiler_params=cp,
      scratch_types=[
          pltpu.VMEM((EPW,), jnp.int32),
          pltpu.VMEM((NCHUNK, C), jnp.int32),
          pltpu.VMEM((C, K), jnp.float32),
          pltpu.VMEM((C, K), jnp.float32),
          pltpu.VMEM_SHARED((NP, K), jnp.float32),
          pltpu.SemaphoreType.DMA,
          pltpu.SemaphoreType.DMA,
          pltpu.SemaphoreType.DMA,
          pltpu.SemaphoreType.DMA,
      ],
  )
  def _msg_kernel(hp_hbm, src_hbm, dst_hbm, z_hbm, out_hbm, src_v, dst_v,
                    rows0, rows1, acc_sh, sem0a, sem0b, sem1a, sem1b):
      c = lax.axis_index("c")
      s = lax.axis_index("s")
      w = c * NS + s
      pltpu.sync_copy(z_hbm, acc_sh.at[pl.ds(s * RPS, RPS)])
      pltpu.sync_copy(src_hbm.at[pl.ds(w * EPW, EPW)], src_v)
      pltpu.sync_copy(dst_hbm.at[w], dst_v)
      plsc.subcore_barrier()

      H = C // 2

      def _gather(i, buf, sa, sb):
          pltpu.async_copy(hp_hbm.at[src_v.at[pl.ds(i * C, H)]],
                           buf.at[pl.ds(0, H)], sa)
          pltpu.async_copy(hp_hbm.at[src_v.at[pl.ds(i * C + H, H)]],
                           buf.at[pl.ds(H, H)], sb)

      def _gwait(buf, sa, sb):
          pltpu.make_async_copy(hp_hbm.at[src_v.at[pl.ds(0, H)]],
                                buf.at[pl.ds(0, H)], sa).wait()
          pltpu.make_async_copy(hp_hbm.at[src_v.at[pl.ds(0, H)]],
                                buf.at[pl.ds(H, H)], sb).wait()

      _gather(0, rows0, sem0a, sem0b)

      def body(j, carry):
          i0 = 2 * j
          i1 = i0 + 1
          _gather(i1, rows1, sem1a, sem1b)
          _gwait(rows0, sem0a, sem0b)
          pltpu.sync_copy(rows0, acc_sh.at[dst_v.at[i0]], add=True)
          _gather(i0 + 2, rows0, sem0a, sem0b)
          _gwait(rows1, sem1a, sem1b)
          pltpu.sync_copy(rows1, acc_sh.at[dst_v.at[i1]], add=True)
          return carry

      lax.fori_loop(0, NCHUNK // 2, body, 0)
      _gwait(rows0, sem0a, sem0b)
      pltpu.sync_copy(rows0, acc_sh.at[dst_v.at[NCHUNK - 1]], add=True)
      plsc.subcore_barrier()
      pltpu.sync_copy(acc_sh.at[pl.ds(s * RPS, RPS)],
                      out_hbm.at[c, pl.ds(s * RPS, RPS)])


  return _msg_kernel


_msg128 = _make_msg_kernel(128, tc_tiling=False)
_msg64 = _make_msg_kernel(64, tc_tiling=False)


def _first_y_body(x_ref, w_ref, degp_ref, y_ref, gamma_ref, beta_ref,
                  tw1_ref, tb1_ref, tw2_ref, tb2_ref,
                  dinv_ref, up_ref, yemb_ref):
    deg = degp_ref[0, 0:N, 0:1] + degp_ref[1, 0:N, 0:1] + 1.0
    dinv = lax.rsqrt(deg)
    dinv_ref[...] = dinv
    u = jnp.dot(x_ref[...], w_ref[...], preferred_element_type=jnp.float32)
    up_ref[0:N, :] = dinv * u
    up_ref[N:NP, :] = jnp.zeros((NP - N, 128), jnp.float32)
    y = y_ref[...]
    mu = jnp.mean(y, axis=0, keepdims=True)
    var = jnp.mean((y - mu) * (y - mu), axis=0, keepdims=True)
    yn = (y - mu) / jnp.sqrt(var + 1e-5) * gamma_ref[...] + beta_ref[...]
    t = jnp.maximum(yn * tw1_ref[...] + tb1_ref[...], 0.0)
    t2 = jnp.dot(t, tw2_ref[...], preferred_element_type=jnp.float32)
    t2 = t2 + tb2_ref[...]
    nrm = jnp.sqrt(jnp.sum(t2 * t2, axis=1, keepdims=True))
    yemb_ref[...] = t2 / jnp.maximum(nrm, 1e-12)


def _mid_body(s_ref, up_ref, dinv_ref, b_ref, w_ref, out_ref):
    dinv = dinv_ref[...]
    z = dinv * (s_ref[0, 0:N] + s_ref[1, 0:N] + up_ref[0:N]) + b_ref[...]
    h = jnp.maximum(z, 0.0)
    u = jnp.dot(h, w_ref[...], preferred_element_type=jnp.float32)
    out_ref[0:N, :] = dinv * u
    out_ref[N:NP, :] = jnp.zeros((NP - N, u.shape[1]), jnp.float32)


def _last_body(s_ref, up_ref, dinv_ref, b_ref, out_ref):
    z = dinv_ref[...] * (s_ref[0, 0:N] + s_ref[1, 0:N] + up_ref[0:N])
    z = z + b_ref[...]
    nrm = jnp.sqrt(jnp.sum(z * z, axis=1, keepdims=True))
    out_ref[...] = z / jnp.maximum(nrm, 1e-12)


def _tc_call(body, out_shapes):
    return pl.pallas_call(body, out_shape=out_shapes)


def kernel(x, y, edge_index, W1, b1, W2, b2, W3, b3, gamma, beta,
           tW1, tb1, tW2, tb2):
    f32 = jnp.float32
    ones128 = jnp.zeros((C, 128), f32).at[:, 0].set(1.0)
    z128 = jnp.zeros((RPS, 128), f32)

    e_src = edge_index[0]
    e_dst = edge_index[1].reshape(NW, NCHUNK, C)

    degp = _deg_kernel(e_dst, ones128, z128)

    dinv, up1, y_emb = _tc_call(_first_y_body, (
        jax.ShapeDtypeStruct((N, 1), f32),
        jax.ShapeDtypeStruct((NP, 128), f32),
        jax.ShapeDtypeStruct((y.shape[0], 64), f32),
    ))(x, W1, degp, y, gamma.reshape(1, 1), beta.reshape(1, 1), tW1,
       tb1.reshape(1, -1), tW2, tb2.reshape(1, -1))

    s1 = _msg128(up1, e_src, e_dst, z128)
    up2 = _tc_call(_mid_body, jax.ShapeDtypeStruct((NP, 128), f32))(
        s1, up1, dinv, b1.reshape(1, -1), W2)

    s2 = _msg128(up2, e_src, e_dst, z128)
    up3 = _tc_call(_mid_body, jax.ShapeDtypeStruct((NP, 64), f32))(
        s2, up2, dinv, b2.reshape(1, -1), W3)

    z64 = jnp.zeros((RPS, 64), f32)
    s3 = _msg64(up3, e_src, e_dst, z64)
    x_emb = _tc_call(_last_body, jax.ShapeDtypeStruct((N, 64), f32))(
        s3, up3, dinv, b3.reshape(1, -1))

    return (x_emb, y_emb)


# speedup vs baseline: 1.1169x; 1.1169x over previous
import functools

import jax
import jax.numpy as jnp
from jax import lax
from jax.experimental import pallas as pl
from jax.experimental.pallas import tpu as pltpu
from jax.experimental.pallas import tpu_sc as plsc

N = 10000
NP = 10240
E = 320000
NC = 2
NS = 16
NW = NC * NS
C = 80
NCHUNK = 125
EPW = NCHUNK * C
RPS = NP // NS

_mesh = plsc.VectorSubcoreMesh(core_axis_name="c", subcore_axis_name="s")


DW = 16


@functools.partial(
    pl.kernel,
    out_type=jax.ShapeDtypeStruct((NC, NP, DW), jnp.float32),
    mesh=_mesh,
    compiler_params=pltpu.CompilerParams(use_tc_tiling_on_sc=False),
    scratch_types=[
        pltpu.VMEM((NCHUNK, C), jnp.int32),
        pltpu.VMEM((C, DW), jnp.float32),
        pltpu.VMEM_SHARED((NP, DW), jnp.float32),
        pltpu.SemaphoreType.DMA,
    ],
)
def _deg_kernel(dst_hbm, ones_hbm, z_hbm, out_hbm, dst_v, ones_v, acc_sh, sem):
    c = lax.axis_index("c")
    s = lax.axis_index("s")
    w = c * NS + s
    pltpu.sync_copy(z_hbm, acc_sh.at[pl.ds(s * RPS, RPS)])
    pltpu.sync_copy(ones_hbm, ones_v)
    pltpu.sync_copy(dst_hbm.at[w], dst_v)
    plsc.subcore_barrier()

    def body(i, carry):
        pltpu.async_copy(ones_v, acc_sh.at[dst_v.at[i]], sem, add=True)
        return carry

    lax.fori_loop(0, NCHUNK, body, 0)

    def drain(i, carry):
        pltpu.make_async_copy(ones_v, acc_sh.at[dst_v.at[0]], sem).wait()
        return carry

    lax.fori_loop(0, NCHUNK, drain, 0)
    plsc.subcore_barrier()
    pltpu.sync_copy(acc_sh.at[pl.ds(s * RPS, RPS)],
                    out_hbm.at[c, pl.ds(s * RPS, RPS)])


def _make_msg_kernel(K, tc_tiling=True):
  cp = pltpu.CompilerParams(use_tc_tiling_on_sc=tc_tiling)

  @functools.partial(
      pl.kernel,
      out_type=jax.ShapeDtypeStruct((NC, NP, K), jnp.float32),
      mesh=_mesh,
      compiler_params=cp,
      scratch_types=[
          pltpu.VMEM((EPW,), jnp.int32),
          pltpu.VMEM((NCHUNK, C), jnp.int32),
          pltpu.VMEM((C, K), jnp.float32),
          pltpu.VMEM((C, K), jnp.float32),
          pltpu.VMEM_SHARED((NP, K), jnp.float32),
          pltpu.SemaphoreType.DMA,
          pltpu.SemaphoreType.DMA,
          pltpu.SemaphoreType.DMA,
          pltpu.SemaphoreType.DMA,
      ],
  )
  def _msg_kernel(hp_hbm, src_hbm, dst_hbm, z_hbm, out_hbm, src_v, dst_v,
                    rows0, rows1, acc_sh, sem0a, sem0b, sem1a, sem1b):
      c = lax.axis_index("c")
      s = lax.axis_index("s")
      w = c * NS + s
      pltpu.sync_copy(z_hbm, acc_sh.at[pl.ds(s * RPS, RPS)])
      pltpu.sync_copy(src_hbm.at[pl.ds(w * EPW, EPW)], src_v)
      pltpu.sync_copy(dst_hbm.at[w], dst_v)
      plsc.subcore_barrier()

      H = C // 2

      def _gather(i, buf, sa, sb):
          pltpu.async_copy(hp_hbm.at[src_v.at[pl.ds(i * C, H)]],
                           buf.at[pl.ds(0, H)], sa)
          pltpu.async_copy(hp_hbm.at[src_v.at[pl.ds(i * C + H, H)]],
                           buf.at[pl.ds(H, H)], sb)

      def _gwait(buf, sa, sb):
          pltpu.make_async_copy(hp_hbm.at[src_v.at[pl.ds(0, H)]],
                                buf.at[pl.ds(0, H)], sa).wait()
          pltpu.make_async_copy(hp_hbm.at[src_v.at[pl.ds(0, H)]],
                                buf.at[pl.ds(H, H)], sb).wait()

      _gather(0, rows0, sem0a, sem0b)

      def body(j, carry):
          i0 = 2 * j
          i1 = i0 + 1
          _gather(i1, rows1, sem1a, sem1b)
          _gwait(rows0, sem0a, sem0b)
          pltpu.sync_copy(rows0, acc_sh.at[dst_v.at[i0]], add=True)
          _gather(i0 + 2, rows0, sem0a, sem0b)
          _gwait(rows1, sem1a, sem1b)
          pltpu.sync_copy(rows1, acc_sh.at[dst_v.at[i1]], add=True)
          return carry

      lax.fori_loop(0, NCHUNK // 2, body, 0)
      _gwait(rows0, sem0a, sem0b)
      pltpu.sync_copy(rows0, acc_sh.at[dst_v.at[NCHUNK - 1]], add=True)
      plsc.subcore_barrier()
      pltpu.sync_copy(acc_sh.at[pl.ds(s * RPS, RPS)],
                      out_hbm.at[c, pl.ds(s * RPS, RPS)])


  return _msg_kernel


_msg128 = _make_msg_kernel(128, tc_tiling=False)
_msg64 = _make_msg_kernel(64, tc_tiling=False)


def _first_y_body(x_ref, w_ref, degp_ref, y_ref, gamma_ref, beta_ref,
                  tw1_ref, tb1_ref, tw2_ref, tb2_ref,
                  dinv_ref, up_ref, yemb_ref):
    deg = degp_ref[0, 0:N, 0:1] + degp_ref[1, 0:N, 0:1] + 1.0
    dinv = lax.rsqrt(deg)
    dinv_ref[...] = dinv
    u = jnp.dot(x_ref[...], w_ref[...], preferred_element_type=jnp.float32)
    up_ref[0:N, :] = dinv * u
    up_ref[N:NP, :] = jnp.zeros((NP - N, 128), jnp.float32)
    y = y_ref[...]
    mu = jnp.mean(y, axis=0, keepdims=True)
    var = jnp.mean((y - mu) * (y - mu), axis=0, keepdims=True)
    yn = (y - mu) / jnp.sqrt(var + 1e-5) * gamma_ref[...] + beta_ref[...]
    t = jnp.maximum(yn * tw1_ref[...] + tb1_ref[...], 0.0)
    t2 = jnp.dot(t, tw2_ref[...], preferred_element_type=jnp.float32)
    t2 = t2 + tb2_ref[...]
    nrm = jnp.sqrt(jnp.sum(t2 * t2, axis=1, keepdims=True))
    yemb_ref[...] = t2 / jnp.maximum(nrm, 1e-12)


def _mid_body(s_ref, up_ref, dinv_ref, b_ref, w_ref, out_ref):
    dinv = dinv_ref[...]
    z = dinv * (s_ref[0, 0:N] + s_ref[1, 0:N] + up_ref[0:N]) + b_ref[...]
    h = jnp.maximum(z, 0.0)
    u = jnp.dot(h, w_ref[...], preferred_element_type=jnp.float32)
    out_ref[0:N, :] = dinv * u
    out_ref[N:NP, :] = jnp.zeros((NP - N, u.shape[1]), jnp.float32)


def _last_body(s_ref, up_ref, dinv_ref, b_ref, out_ref):
    z = dinv_ref[...] * (s_ref[0, 0:N] + s_ref[1, 0:N] + up_ref[0:N])
    z = z + b_ref[...]
    nrm = jnp.sqrt(jnp.sum(z * z, axis=1, keepdims=True))
    out_ref[...] = z / jnp.maximum(nrm, 1e-12)


def _tc_call(body, out_shapes):
    return pl.pallas_call(body, out_shape=out_shapes)


def kernel(x, y, edge_index, W1, b1, W2, b2, W3, b3, gamma, beta,
           tW1, tb1, tW2, tb2):
    f32 = jnp.float32
    ones_dw = jnp.zeros((C, DW), f32).at[:, 0].set(1.0)
    zdw = jnp.zeros((RPS, DW), f32)
    z128 = jnp.zeros((RPS, 128), f32)

    e_src = edge_index[0]
    e_dst = edge_index[1].reshape(NW, NCHUNK, C)

    degp = _deg_kernel(e_dst, ones_dw, zdw)

    dinv, up1, y_emb = _tc_call(_first_y_body, (
        jax.ShapeDtypeStruct((N, 1), f32),
        jax.ShapeDtypeStruct((NP, 128), f32),
        jax.ShapeDtypeStruct((y.shape[0], 64), f32),
    ))(x, W1, degp, y, gamma.reshape(1, 1), beta.reshape(1, 1), tW1,
       tb1.reshape(1, -1), tW2, tb2.reshape(1, -1))

    s1 = _msg128(up1, e_src, e_dst, z128)
    up2 = _tc_call(_mid_body, jax.ShapeDtypeStruct((NP, 128), f32))(
        s1, up1, dinv, b1.reshape(1, -1), W2)

    s2 = _msg128(up2, e_src, e_dst, z128)
    up3 = _tc_call(_mid_body, jax.ShapeDtypeStruct((NP, 64), f32))(
        s2, up2, dinv, b2.reshape(1, -1), W3)

    z64 = jnp.zeros((RPS, 64), f32)
    s3 = _msg64(up3, e_src, e_dst, z64)
    x_emb = _tc_call(_last_body, jax.ShapeDtypeStruct((N, 64), f32))(
        s3, up3, dinv, b3.reshape(1, -1))

    return (x_emb, y_emb)
